# baseline (speedup 1.0000x reference)
import functools

import jax
import jax.numpy as jnp
from jax import lax
from jax.experimental import pallas as pl
from jax.experimental.pallas import tpu as pltpu
from jax.experimental.pallas import tpu_sc as plsc

_SUP = 300
_EPS = 0.001
_ROW = 2 * _SUP + 1
_N = 65536

_NC = 2
_NS = 16
_NW = _NC * _NS
_RPW = _N // _NW
_GRP = 16
_NGRP = _RPW // _GRP
_GFLAT = _GRP * _ROW
_NBUF = 4


def _prep_kernel(x_ref, s_ref, p_ref, a_ref, b_ref):
    x = x_ref[:]
    enc = jnp.sign(x) * (jnp.sqrt(jnp.abs(x) + 1.0) - 1.0) + _EPS * x
    enc = jnp.clip(enc, -float(_SUP), float(_SUP))
    fl = jnp.floor(enc)
    rem = enc - fl
    fli = fl.astype(jnp.int32)
    top = fli >= _SUP
    s_ref[:] = enc
    p_ref[:] = jnp.where(top, 2 * _SUP - 1, fli + _SUP)
    a_ref[:] = jnp.where(top, 0.0, 1.0 - rem)
    b_ref[:] = jnp.where(top, 1.0, rem)


def _sc_expand(p_hbm, a_hbm, b_hbm, out_hbm, p_v, a_v, b_v, bufs, insem, outsem):
    wid = lax.axis_index("s") * _NC + lax.axis_index("c")
    base_row = wid * _RPW
    pltpu.async_copy(p_hbm.at[pl.ds(base_row, _RPW)], p_v, insem)
    pltpu.async_copy(a_hbm.at[pl.ds(base_row, _RPW)], a_v, insem)
    pltpu.async_copy(b_hbm.at[pl.ds(base_row, _RPW)], b_v, insem).wait()
    pltpu.make_async_copy(a_hbm.at[pl.ds(base_row, _RPW)], a_v, insem).wait()
    pltpu.make_async_copy(p_hbm.at[pl.ds(base_row, _RPW)], p_v, insem).wait()

    zeros16 = jnp.zeros((_GRP,), jnp.float32)
    lane = lax.iota(jnp.int32, _GRP)

    def _zero_body(i, _):
        for b in range(_NBUF):
            bufs[b][pl.ds(i * _GRP, _GRP)] = zeros16
        return 0

    lax.fori_loop(0, _GFLAT // _GRP, _zero_body, 0)

    def _scatter_group(slot, g):
        p16 = p_v[pl.ds(g * _GRP, _GRP)]
        idx = lane * _ROW + p16
        plsc.store_scatter(bufs[slot], [idx], a_v[pl.ds(g * _GRP, _GRP)])
        plsc.store_scatter(bufs[slot], [idx + 1], b_v[pl.ds(g * _GRP, _GRP)])

    def _start_dma(slot, g):
        flat = (base_row + g * _GRP) * _ROW
        pltpu.make_async_copy(
            bufs[slot], out_hbm.at[pl.ds(flat, _GFLAT)], outsem
        ).start()

    def _wait_and_clear(slot, g_old):
        flat = (base_row + g_old * _GRP) * _ROW
        pltpu.make_async_copy(
            bufs[slot], out_hbm.at[pl.ds(flat, _GFLAT)], outsem
        ).wait()
        p16 = p_v[pl.ds(g_old * _GRP, _GRP)]
        idx = lane * _ROW + p16
        plsc.store_scatter(bufs[slot], [idx], zeros16)
        plsc.store_scatter(bufs[slot], [idx + 1], zeros16)

    for b in range(_NBUF):
        _scatter_group(b, jnp.int32(b))
        _start_dma(b, jnp.int32(b))

    def _main_body(o, _):
        for b in range(_NBUF):
            g = _NBUF + o * _NBUF + b
            _wait_and_clear(b, g - _NBUF)
            _scatter_group(b, g)
            _start_dma(b, g)
        return 0

    lax.fori_loop(0, (_NGRP - _NBUF) // _NBUF, _main_body, 0)

    for b in range(_NBUF):
        g_old = jnp.int32(_NGRP - _NBUF + b)
        flat = (base_row + g_old * _GRP) * _ROW
        pltpu.make_async_copy(
            bufs[b], out_hbm.at[pl.ds(flat, _GFLAT)], outsem
        ).wait()


@jax.jit
def kernel(x):
    n = x.shape[0]
    x2 = x.reshape(512, 128)
    enc_s, p, a, b = pl.pallas_call(
        _prep_kernel,
        out_shape=[
            jax.ShapeDtypeStruct((512, 128), jnp.float32),
            jax.ShapeDtypeStruct((512, 128), jnp.int32),
            jax.ShapeDtypeStruct((512, 128), jnp.float32),
            jax.ShapeDtypeStruct((512, 128), jnp.float32),
        ],
    )(x2)

    sc = pl.kernel(
        _sc_expand,
        out_type=jax.ShapeDtypeStruct((n * _ROW,), jnp.float32),
        mesh=plsc.VectorSubcoreMesh(core_axis_name="c", subcore_axis_name="s"),
        scratch_types=[
            pltpu.VMEM((_RPW,), jnp.int32),
            pltpu.VMEM((_RPW,), jnp.float32),
            pltpu.VMEM((_RPW,), jnp.float32),
            [pltpu.VMEM((_GFLAT,), jnp.float32) for _ in range(_NBUF)],
            pltpu.SemaphoreType.DMA,
            pltpu.SemaphoreType.DMA,
        ],
        compiler_params=pltpu.CompilerParams(needs_layout_passes=False),
    )
    enc_v = sc(p.reshape(n), a.reshape(n), b.reshape(n)).reshape(n, _ROW)
    return (enc_s.reshape(n), enc_v)
